# Initial kernel scaffold; baseline (speedup 1.0000x reference)
import jax
import jax.numpy as jnp
from jax.experimental import pallas as pl


def kernel(h_u, edge_index, edge_type, edge_dir, Basis, alpha, W_O, W_I, W_S, W_rel):
    raise NotImplementedError("write your pallas kernel here")


# speedup vs baseline: 4.9176x; 4.9176x over previous
import functools

import jax
import jax.numpy as jnp
from jax import lax
from jax.experimental import pallas as pl
from jax.experimental.pallas import tpu as pltpu
from jax.experimental.pallas import tpu_sc as plsc

N = 10000
E = 320000
D = 128
NUM_REL = 2
NUM_LAYERS = 3

NC = 2
NS = 16
NW = NC * NS
CH = 128
KCH = -(-E // (NW * CH))
E_PAD = NW * CH * KCH
NP = 10240
ZR = 64
ZERO_PER_TILE = NP // NS
OUT_UNIT = 8
OUT_UNITS = N // OUT_UNIT
OUT_UNITS_PER_TILE = -(-OUT_UNITS // NS)
BN = 1000

_f32 = jnp.float32


def _rel_tables(alpha, Basis, W_O, W_I, W_S, W_rel):
    def body(a_ref, b_ref, wo_ref, wi_ref, ws_ref, wr_ref, t_ref):
        hr = jnp.dot(a_ref[...], b_ref[...], preferred_element_type=_f32)
        for l in range(NUM_LAYERS):
            t_o = jnp.dot(hr[0:2], wo_ref[l], preferred_element_type=_f32)
            t_i = jnp.dot(hr[0:2], wi_ref[l], preferred_element_type=_f32)
            t_s = jnp.dot(hr[2:3], ws_ref[l], preferred_element_type=_f32)
            t_ref[l] = jnp.concatenate(
                [t_o, t_i, t_s, jnp.zeros((3, D), _f32)], axis=0)
            hr = jnp.dot(hr, wr_ref[l], preferred_element_type=_f32)
            if l < NUM_LAYERS - 1:
                hr = jax.nn.relu(hr)

    return pl.pallas_call(
        body,
        out_shape=jax.ShapeDtypeStruct((NUM_LAYERS, 8, D), _f32),
    )(alpha, Basis, W_O, W_I, W_S, W_rel)


def _dense_tail(h, t_ref, wo_ref, wi_ref, ws_ref, x4_ref, ss_ref):
    xo = jnp.dot(h, wo_ref[...], preferred_element_type=_f32)
    xi = jnp.dot(h, wi_ref[...], preferred_element_type=_f32)
    xs = jnp.dot(h, ws_ref[...], preferred_element_type=_f32)
    t = t_ref[...]
    x4_ref[0] = xo - t[0:1]
    x4_ref[1] = xo - t[1:2]
    x4_ref[2] = xi - t[2:3]
    x4_ref[3] = xi - t[3:4]
    ss_ref[...] = xs - t[4:5]


_W_SPEC = pl.BlockSpec((D, D), lambda i: (0, 0))
_T_SPEC = pl.BlockSpec((8, D), lambda i: (0, 0))
_ROW_SPEC = pl.BlockSpec((BN, D), lambda i: (i, 0))
_X4_SPEC = pl.BlockSpec((4, BN, D), lambda i: (0, i, 0))
_AGG_SPEC = pl.BlockSpec((NC, BN, D), lambda i: (0, i, 0))
_DENSE_OUT = [
    jax.ShapeDtypeStruct((4, N, D), _f32),
    jax.ShapeDtypeStruct((N, D), _f32),
]


def _dense_first(h_u, t_l, Wo, Wi, Ws):
    def body(h_ref, t_ref, wo_ref, wi_ref, ws_ref, x4_ref, ss_ref):
        _dense_tail(h_ref[...], t_ref, wo_ref, wi_ref, ws_ref, x4_ref, ss_ref)

    return pl.pallas_call(
        body,
        grid=(N // BN,),
        in_specs=[_ROW_SPEC, _T_SPEC, _W_SPEC, _W_SPEC, _W_SPEC],
        out_specs=[_X4_SPEC, _ROW_SPEC],
        out_shape=_DENSE_OUT,
    )(h_u, t_l, Wo, Wi, Ws)


def _dense_next(ss_prev, agg, t_l, Wo, Wi, Ws):
    def body(ssp_ref, agg_ref, t_ref, wo_ref, wi_ref, ws_ref, x4_ref, ss_ref):
        h = jax.nn.relu(ssp_ref[...] + agg_ref[0] + agg_ref[1])
        _dense_tail(h, t_ref, wo_ref, wi_ref, ws_ref, x4_ref, ss_ref)

    return pl.pallas_call(
        body,
        grid=(N // BN,),
        in_specs=[_ROW_SPEC, _AGG_SPEC, _T_SPEC, _W_SPEC, _W_SPEC, _W_SPEC],
        out_specs=[_X4_SPEC, _ROW_SPEC],
        out_shape=_DENSE_OUT,
    )(ss_prev, agg, t_l, Wo, Wi, Ws)


def _combine(ss, agg):
    def body(ss_ref, agg_ref, out_ref):
        out_ref[...] = ss_ref[...] + agg_ref[0] + agg_ref[1]

    return pl.pallas_call(
        body,
        grid=(N // BN,),
        in_specs=[_ROW_SPEC, _AGG_SPEC],
        out_specs=_ROW_SPEC,
        out_shape=jax.ShapeDtypeStruct((N, D), _f32),
    )(ss, agg)


_MESH = plsc.VectorSubcoreMesh(core_axis_name="c", subcore_axis_name="s")


@functools.partial(
    pl.kernel,
    mesh=_MESH,
    out_type=jax.ShapeDtypeStruct((NC, N, D), _f32),
    scratch_types=[
        pltpu.VMEM((KCH, CH), jnp.int32),
        pltpu.VMEM((KCH, CH), jnp.int32),
        pltpu.VMEM((CH, D), _f32),
        pltpu.VMEM((ZR, D), _f32),
        pltpu.VMEM_SHARED((NP, D), _f32),
        pltpu.SemaphoreType.DMA,
    ],
)
def _sc_edge_pass(x4_hbm, idx_hbm, dst_hbm, out_hbm,
                  idx_v, dst_v, rows_v, zbuf, aggsh, gsem):
    cid = lax.axis_index("c")
    sid = lax.axis_index("s")
    wid = sid * NC + cid

    def _z(i, c):
        r = i // (D // 16)
        col = (i % (D // 16)) * 16
        zbuf[r, pl.ds(col, 16)] = jnp.zeros((16,), _f32)
        return c

    lax.fori_loop(0, ZR * (D // 16), _z, 0)

    def _zs(i, c):
        pltpu.sync_copy(zbuf, aggsh.at[pl.ds(sid * ZERO_PER_TILE + i * ZR, ZR)])
        return c

    lax.fori_loop(0, ZERO_PER_TILE // ZR, _zs, 0)

    plsc.subcore_barrier()

    pltpu.sync_copy(idx_hbm.at[wid], idx_v)
    pltpu.sync_copy(dst_hbm.at[wid], dst_v)

    def _chunk(j, c):
        pltpu.async_copy(x4_hbm.at[idx_v.at[j]], rows_v, gsem).wait()
        pltpu.sync_copy(rows_v, aggsh.at[dst_v.at[j]], add=True)
        return c

    lax.fori_loop(0, KCH, _chunk, 0)

    plsc.subcore_barrier()

    def _co(u, c):
        unit = sid * OUT_UNITS_PER_TILE + u

        @pl.when(unit < OUT_UNITS)
        def _():
            base = unit * OUT_UNIT
            pltpu.sync_copy(aggsh.at[pl.ds(base, OUT_UNIT)],
                            out_hbm.at[cid, pl.ds(base, OUT_UNIT)])

        return c

    lax.fori_loop(0, OUT_UNITS_PER_TILE, _co, 0)


def kernel(h_u, edge_index, edge_type, edge_dir, Basis, alpha, W_O, W_I, W_S, W_rel):
    src = edge_index[0]
    dst = edge_index[1]
    gidx = (edge_dir * 2 + edge_type) * N + src
    pad = E_PAD - E
    gidx = jnp.concatenate([gidx.astype(jnp.int32),
                            jnp.zeros((pad,), jnp.int32)])
    dstp = jnp.concatenate([dst.astype(jnp.int32),
                            jnp.full((pad,), N, jnp.int32)])
    idx3 = gidx.reshape(NW, KCH, CH)
    dst3 = dstp.reshape(NW, KCH, CH)

    t_all = _rel_tables(alpha, Basis, W_O, W_I, W_S, W_rel)

    x4, ss = _dense_first(h_u, t_all[0], W_O[0], W_I[0], W_S[0])
    for l in range(1, NUM_LAYERS):
        agg = _sc_edge_pass(x4.reshape(4 * N, D), idx3, dst3)
        x4, ss = _dense_next(ss, agg, t_all[l], W_O[l], W_I[l], W_S[l])
    agg = _sc_edge_pass(x4.reshape(4 * N, D), idx3, dst3)
    return _combine(ss, agg)
